# baseline (speedup 1.0000x reference)
import functools

import jax
import jax.numpy as jnp
from jax import lax
from jax.experimental import pallas as pl
from jax.experimental.pallas import tpu as pltpu
from jax.experimental.pallas import tpu_sc as plsc

f32 = jnp.float32
i32 = jnp.int32

N_NODES = 10000
D = 128
NROW = 80
NPAD = NROW * 128
NC, NS = 2, 16
NW = NC * NS
NCH = 81
EPT = NCH * 128
EPAD = NW * EPT
G = 3
NGRP = NCH // G
OUT_PT = NPAD // NS
ROWB = 1000
NBLK = N_NODES // ROWB


def _tc1_body(x_ref, w_ref, att_ref, h_ref, ab_ref, mx_ref):
    i = pl.program_id(0)
    h = jnp.dot(x_ref[...], w_ref[...], preferred_element_type=f32)
    h_ref[...] = h
    ab = jnp.dot(h, att_ref[...], preferred_element_type=f32)
    ab_ref[...] = ab

    @pl.when(i == 0)
    def _():
        mx_ref[...] = jnp.full((8, 128), -jnp.inf, f32)

    bm = jnp.broadcast_to(jnp.max(ab, axis=0, keepdims=True), (8, 128))
    mx_ref[...] = jnp.maximum(mx_ref[...], bm)


_tc1 = pl.pallas_call(
    _tc1_body,
    grid=(NBLK,),
    in_specs=[
        pl.BlockSpec((ROWB, D), lambda i: (i, 0)),
        pl.BlockSpec((D, D), lambda i: (0, 0)),
        pl.BlockSpec((D, D), lambda i: (0, 0)),
    ],
    out_specs=[
        pl.BlockSpec((ROWB, D), lambda i: (i, 0)),
        pl.BlockSpec((ROWB, D), lambda i: (i, 0)),
        pl.BlockSpec((8, 128), lambda i: (0, 0)),
    ],
    out_shape=[
        jax.ShapeDtypeStruct((N_NODES, D), f32),
        jax.ShapeDtypeStruct((N_NODES, D), f32),
        jax.ShapeDtypeStruct((8, 128), f32),
    ],
)


def _sc_mesh():
    return plsc.VectorSubcoreMesh(
        core_axis_name="c", subcore_axis_name="s", num_cores=NC, num_subcores=NS
    )


@functools.partial(
    pl.kernel,
    out_type=(
        jax.ShapeDtypeStruct((NW, NCH, 128), f32),
        jax.ShapeDtypeStruct((NC, NROW, 128), f32),
    ),
    mesh=_sc_mesh(),
    compiler_params=pltpu.CompilerParams(needs_layout_passes=False, use_tc_tiling_on_sc=False),
    scratch_types=[
        pltpu.VMEM((NPAD,), f32),
        pltpu.VMEM((NPAD,), f32),
        pltpu.VMEM((NCH, 128), i32),
        pltpu.VMEM((NCH, 128), i32),
        pltpu.VMEM((NCH, 128), f32),
        pltpu.VMEM((NROW, 128), f32),
        pltpu.VMEM((16,), f32),
        pltpu.VMEM((NROW,), i32),
        pltpu.VMEM_SHARED((NROW, 128), f32),
    ],
)
def _sc_edge_softmax(asrc_h, adst_h, src_h, dst_h, m_h, ex_h, den_h,
                     asrc_v, adst_v, srcv, dstv, exv, denv, mv, idxv, dsh):
    cid = lax.axis_index("c")
    sid = lax.axis_index("s")
    wid = cid * NS + sid
    pltpu.sync_copy(asrc_h, asrc_v)
    pltpu.sync_copy(adst_h, adst_v)
    pltpu.sync_copy(m_h, mv)
    pltpu.sync_copy(src_h.at[wid], srcv)
    pltpu.sync_copy(dst_h.at[wid], dstv)

    zero16 = jnp.zeros((16,), f32)

    def zero_body(r, carry):
        for k in range(8):
            denv[r, pl.ds(k * 16, 16)] = zero16
        return carry

    lax.fori_loop(0, NROW, zero_body, 0)
    for i in range(NROW // 16):
        idxv[pl.ds(i * 16, 16)] = jnp.arange(16, dtype=i32) + (i * 16)

    @pl.when(sid == 0)
    def _():
        pltpu.sync_copy(denv, dsh)

    plsc.subcore_barrier()
    mvec = mv[...]

    def chunk(j, carry):
        for k in range(8):
            s16 = srcv[j, pl.ds(k * 16, 16)]
            d16 = dstv[j, pl.ds(k * 16, 16)]
            a = plsc.load_gather(asrc_v, [s16]) + plsc.load_gather(adst_v, [d16])
            a = jnp.where(a >= 0.0, a, 0.2 * a)
            ex = jnp.exp(a - mvec)
            exv[j, pl.ds(k * 16, 16)] = ex
            plsc.addupdate_scatter(
                denv, [lax.shift_right_logical(d16, 7), d16 & 127], ex
            )
        return carry

    lax.fori_loop(0, NCH, chunk, 0)
    pltpu.sync_copy(exv, ex_h.at[wid])
    pltpu.sync_copy(denv, dsh.at[idxv], add=True)
    plsc.subcore_barrier()

    @pl.when(sid == 0)
    def _():
        pltpu.sync_copy(dsh, den_h.at[cid])


@functools.partial(
    pl.kernel,
    out_type=jax.ShapeDtypeStruct((NC, NPAD, 128), f32),
    mesh=_sc_mesh(),
    compiler_params=pltpu.CompilerParams(needs_layout_passes=False, use_tc_tiling_on_sc=False),
    scratch_types=[
        pltpu.VMEM((NROW, 128), f32),
        pltpu.VMEM((8, 128), f32),
        pltpu.VMEM((3, 128), i32),
        pltpu.VMEM((3, 128), i32),
        pltpu.VMEM((3, 128), f32),
        pltpu.VMEM((128, 128), f32),
        pltpu.VMEM_SHARED((NPAD, 128), f32),
        pltpu.SemaphoreType.DMA,
    ],
)
def _sc_scatter(src_h, dst_h, ex_h, den_h, hp_h, zeros_h, outp_h,
                denv, tmpv, srcg, dstg, cfg, rows, outsh, gsem0):
    cid = lax.axis_index("c")
    sid = lax.axis_index("s")
    wid = cid * NS + sid
    row0 = sid * OUT_PT
    pltpu.sync_copy(zeros_h.at[pl.ds(row0, OUT_PT)],
                    outsh.at[pl.ds(row0, OUT_PT)])

    pltpu.sync_copy(den_h.at[0], denv)

    def addb(s, carry):
        pltpu.sync_copy(den_h.at[1].at[pl.ds(s * 8, 8)], tmpv)
        for r in range(8):
            for k in range(8):
                denv[s * 8 + r, pl.ds(k * 16, 16)] = (
                    denv[s * 8 + r, pl.ds(k * 16, 16)]
                    + tmpv[r, pl.ds(k * 16, 16)]
                )
        return carry

    lax.fori_loop(0, NROW // 8, addb, 0)
    plsc.subcore_barrier()

    def group(g, carry):
        pltpu.sync_copy(src_h.at[wid].at[pl.ds(g * 3, 3)], srcg)
        pltpu.sync_copy(dst_h.at[wid].at[pl.ds(g * 3, 3)], dstg)
        pltpu.sync_copy(ex_h.at[wid].at[pl.ds(g * 3, 3)], cfg)
        for cc in range(3):
            for k in range(8):
                d16 = dstg[cc, pl.ds(k * 16, 16)]
                den16 = plsc.load_gather(
                    denv, [lax.shift_right_logical(d16, 7), d16 & 127]
                )
                cfg[cc, pl.ds(k * 16, 16)] = (
                    cfg[cc, pl.ds(k * 16, 16)] / (den16 + 1e-16)
                )
            pltpu.async_copy(hp_h.at[srcg.at[cc]], rows, gsem0).wait()

            def scale(e, c2):
                cf = plsc.load_gather(cfg.at[cc], [jnp.full((16,), e, i32)])
                for q in range(8):
                    rows[e, pl.ds(q * 16, 16)] = (
                        rows[e, pl.ds(q * 16, 16)] * cf
                    )
                return c2

            lax.fori_loop(0, 128, scale, 0)
            pltpu.sync_copy(rows, outsh.at[dstg.at[cc]], add=True)
        return carry

    lax.fori_loop(0, NGRP, group, 0)
    plsc.subcore_barrier()
    pltpu.sync_copy(outsh.at[pl.ds(row0, OUT_PT)],
                    outp_h.at[cid].at[pl.ds(row0, OUT_PT)])


def _tc2_body(p0_ref, p1_ref, b_ref, o_ref):
    v = p0_ref[...] + p1_ref[...] + b_ref[...]
    o_ref[...] = jnp.where(v > 0.0, v, jnp.exp(jnp.minimum(v, 0.0)) - 1.0)


_tc2 = pl.pallas_call(
    _tc2_body,
    grid=(NBLK,),
    in_specs=[
        pl.BlockSpec((ROWB, D), lambda i: (i, 0)),
        pl.BlockSpec((ROWB, D), lambda i: (i, 0)),
        pl.BlockSpec((1, D), lambda i: (0, 0)),
    ],
    out_specs=pl.BlockSpec((ROWB, D), lambda i: (i, 0)),
    out_shape=jax.ShapeDtypeStruct((N_NODES, D), f32),
)


@jax.jit
def kernel(x, edge_index, W, att_src, att_dst, bias):
    n = x.shape[0]
    e = edge_index.shape[1]
    loop = jnp.arange(n, dtype=i32)
    pad = jnp.full((EPAD - e - n,), n, dtype=i32)
    src3 = jnp.concatenate([edge_index[0], loop, pad]).reshape(NW, NCH, 128)
    dst3 = jnp.concatenate([edge_index[1], loop, pad]).reshape(NW, NCH, 128)

    att2 = (
        jnp.zeros((D, D), f32).at[:, 0].set(att_src).at[:, 1].set(att_dst)
    )
    h, ab, mx = _tc1(x, W, att2)

    a_src_p = jnp.pad(ab[:, 0], (0, NPAD - n))
    a_dst_p = jnp.pad(ab[:, 1], (0, NPAD - n))
    m = mx[0, 0] + mx[0, 1]
    m = jnp.where(m > 0.0, m, 0.2 * m)
    m16 = jnp.full((16,), m, f32)

    ex3, dens = _sc_edge_softmax(a_src_p, a_dst_p, src3, dst3, m16)

    h_pad = jnp.pad(h, ((0, NPAD - n), (0, 0)))
    zeros = jnp.zeros((NPAD, D), f32)
    outp = _sc_scatter(src3, dst3, ex3, dens, h_pad, zeros)

    return _tc2(outp[0, :n], outp[1, :n], bias.reshape(1, D))


# speedup vs baseline: 1.0166x; 1.0166x over previous
import functools

import jax
import jax.numpy as jnp
from jax import lax
from jax.experimental import pallas as pl
from jax.experimental.pallas import tpu as pltpu
from jax.experimental.pallas import tpu_sc as plsc

f32 = jnp.float32
i32 = jnp.int32

N_NODES = 10000
D = 128
NROW = 80
NPAD = NROW * 128
NC, NS = 2, 16
NW = NC * NS
NCH = 81
EPT = NCH * 128
EPAD = NW * EPT
G = 3
NGRP = NCH // G
OUT_PT = NPAD // NS
ROWB = 1000
NBLK = N_NODES // ROWB


def _tc1_body(x_ref, w_ref, att_ref, h_ref, ab_ref, mx_ref):
    i = pl.program_id(0)
    h = jnp.dot(x_ref[...], w_ref[...], preferred_element_type=f32)
    h_ref[...] = h
    ab = jnp.dot(h, att_ref[...], preferred_element_type=f32)
    ab_ref[...] = ab

    @pl.when(i == 0)
    def _():
        mx_ref[...] = jnp.full((8, 128), -jnp.inf, f32)

    bm = jnp.broadcast_to(jnp.max(ab, axis=0, keepdims=True), (8, 128))
    mx_ref[...] = jnp.maximum(mx_ref[...], bm)


_tc1 = pl.pallas_call(
    _tc1_body,
    grid=(NBLK,),
    in_specs=[
        pl.BlockSpec((ROWB, D), lambda i: (i, 0)),
        pl.BlockSpec((D, D), lambda i: (0, 0)),
        pl.BlockSpec((D, D), lambda i: (0, 0)),
    ],
    out_specs=[
        pl.BlockSpec((ROWB, D), lambda i: (i, 0)),
        pl.BlockSpec((ROWB, D), lambda i: (i, 0)),
        pl.BlockSpec((8, 128), lambda i: (0, 0)),
    ],
    out_shape=[
        jax.ShapeDtypeStruct((N_NODES, D), f32),
        jax.ShapeDtypeStruct((N_NODES, D), f32),
        jax.ShapeDtypeStruct((8, 128), f32),
    ],
)


def _sc_mesh():
    return plsc.VectorSubcoreMesh(
        core_axis_name="c", subcore_axis_name="s", num_cores=NC, num_subcores=NS
    )


@functools.partial(
    pl.kernel,
    out_type=(
        jax.ShapeDtypeStruct((NW, NCH, 128), f32),
        jax.ShapeDtypeStruct((NC, NROW, 128), f32),
    ),
    mesh=_sc_mesh(),
    compiler_params=pltpu.CompilerParams(needs_layout_passes=False, use_tc_tiling_on_sc=False),
    scratch_types=[
        pltpu.VMEM((NPAD,), f32),
        pltpu.VMEM((NPAD,), f32),
        pltpu.VMEM((NCH, 128), i32),
        pltpu.VMEM((NCH, 128), i32),
        pltpu.VMEM((NCH, 128), f32),
        pltpu.VMEM((NROW, 128), f32),
        pltpu.VMEM((16,), f32),
        pltpu.VMEM((NROW,), i32),
        pltpu.VMEM_SHARED((NROW, 128), f32),
    ],
)
def _sc_edge_softmax(asrc_h, adst_h, src_h, dst_h, m_h, ex_h, den_h,
                     asrc_v, adst_v, srcv, dstv, exv, denv, mv, idxv, dsh):
    cid = lax.axis_index("c")
    sid = lax.axis_index("s")
    wid = cid * NS + sid
    pltpu.sync_copy(asrc_h, asrc_v)
    pltpu.sync_copy(adst_h, adst_v)
    pltpu.sync_copy(m_h, mv)
    pltpu.sync_copy(src_h.at[wid], srcv)
    pltpu.sync_copy(dst_h.at[wid], dstv)

    zero16 = jnp.zeros((16,), f32)

    def zero_body(r, carry):
        for k in range(8):
            denv[r, pl.ds(k * 16, 16)] = zero16
        return carry

    lax.fori_loop(0, NROW, zero_body, 0)
    for i in range(NROW // 16):
        idxv[pl.ds(i * 16, 16)] = jnp.arange(16, dtype=i32) + (i * 16)

    @pl.when(sid == 0)
    def _():
        pltpu.sync_copy(denv, dsh)

    plsc.subcore_barrier()
    mvec = mv[...]

    def chunk(j, carry):
        for k in range(8):
            s16 = srcv[j, pl.ds(k * 16, 16)]
            d16 = dstv[j, pl.ds(k * 16, 16)]
            a = plsc.load_gather(asrc_v, [s16]) + plsc.load_gather(adst_v, [d16])
            a = jnp.where(a >= 0.0, a, 0.2 * a)
            ex = jnp.exp(a - mvec)
            exv[j, pl.ds(k * 16, 16)] = ex
            plsc.addupdate_scatter(
                denv, [lax.shift_right_logical(d16, 7), d16 & 127], ex
            )
        return carry

    lax.fori_loop(0, NCH, chunk, 0)
    pltpu.sync_copy(exv, ex_h.at[wid])
    pltpu.sync_copy(denv, dsh.at[idxv], add=True)
    plsc.subcore_barrier()

    @pl.when(sid == 0)
    def _():
        pltpu.sync_copy(dsh, den_h.at[cid])


@functools.partial(
    pl.kernel,
    out_type=jax.ShapeDtypeStruct((NC, NPAD, 128), f32),
    mesh=_sc_mesh(),
    compiler_params=pltpu.CompilerParams(needs_layout_passes=False, use_tc_tiling_on_sc=False),
    scratch_types=[
        pltpu.VMEM((NROW, 128), f32),
        pltpu.VMEM((8, 128), f32),
        pltpu.VMEM((3, 128), i32),
        pltpu.VMEM((3, 128), i32),
        pltpu.VMEM((3, 128), f32),
        pltpu.VMEM((128, 128), f32),
        pltpu.VMEM_SHARED((NPAD, 128), f32),
        pltpu.SemaphoreType.DMA,
    ],
)
def _sc_scatter(src_h, dst_h, ex_h, den_h, hp_h, zeros_h, outp_h,
                denv, tmpv, srcg, dstg, cfg, rows, outsh, gsem0):
    cid = lax.axis_index("c")
    sid = lax.axis_index("s")
    wid = cid * NS + sid
    row0 = sid * OUT_PT
    pltpu.sync_copy(zeros_h.at[pl.ds(row0, OUT_PT)],
                    outsh.at[pl.ds(row0, OUT_PT)])

    pltpu.sync_copy(den_h.at[0], denv)

    def addb(s, carry):
        pltpu.sync_copy(den_h.at[1].at[pl.ds(s * 8, 8)], tmpv)
        for r in range(8):
            for k in range(8):
                denv[s * 8 + r, pl.ds(k * 16, 16)] = (
                    denv[s * 8 + r, pl.ds(k * 16, 16)]
                    + tmpv[r, pl.ds(k * 16, 16)]
                )
        return carry

    lax.fori_loop(0, NROW // 8, addb, 0)
    plsc.subcore_barrier()

    def group(g, carry):
        pltpu.sync_copy(src_h.at[wid].at[pl.ds(g * 3, 3)], srcg)
        pltpu.sync_copy(dst_h.at[wid].at[pl.ds(g * 3, 3)], dstg)
        pltpu.sync_copy(ex_h.at[wid].at[pl.ds(g * 3, 3)], cfg)
        for cc in range(3):
            for k in range(8):
                d16 = dstg[cc, pl.ds(k * 16, 16)]
                den16 = plsc.load_gather(
                    denv, [lax.shift_right_logical(d16, 7), d16 & 127]
                )
                cfg[cc, pl.ds(k * 16, 16)] = (
                    cfg[cc, pl.ds(k * 16, 16)] / (den16 + 1e-16)
                )
            pltpu.async_copy(hp_h.at[srcg.at[cc]], rows, gsem0).wait()

            def scale(e2, c2):
                for u in range(2):
                    e = e2 * 2 + u
                    cf = plsc.load_gather(
                        cfg.at[cc], [jnp.full((16,), e, i32)])
                    for q in range(8):
                        rows[e, pl.ds(q * 16, 16)] = (
                            rows[e, pl.ds(q * 16, 16)] * cf
                        )
                return c2

            lax.fori_loop(0, 64, scale, 0)
            pltpu.sync_copy(rows, outsh.at[dstg.at[cc]], add=True)
        return carry

    lax.fori_loop(0, NGRP, group, 0)
    plsc.subcore_barrier()
    pltpu.sync_copy(outsh.at[pl.ds(row0, OUT_PT)],
                    outp_h.at[cid].at[pl.ds(row0, OUT_PT)])


def _tc2_body(p0_ref, p1_ref, b_ref, o_ref):
    v = p0_ref[...] + p1_ref[...] + b_ref[...]
    o_ref[...] = jnp.where(v > 0.0, v, jnp.exp(jnp.minimum(v, 0.0)) - 1.0)


_tc2 = pl.pallas_call(
    _tc2_body,
    grid=(NBLK,),
    in_specs=[
        pl.BlockSpec((ROWB, D), lambda i: (i, 0)),
        pl.BlockSpec((ROWB, D), lambda i: (i, 0)),
        pl.BlockSpec((1, D), lambda i: (0, 0)),
    ],
    out_specs=pl.BlockSpec((ROWB, D), lambda i: (i, 0)),
    out_shape=jax.ShapeDtypeStruct((N_NODES, D), f32),
)


@jax.jit
def kernel(x, edge_index, W, att_src, att_dst, bias):
    n = x.shape[0]
    e = edge_index.shape[1]
    loop = jnp.arange(n, dtype=i32)
    pad = jnp.full((EPAD - e - n,), n, dtype=i32)
    src3 = jnp.concatenate([edge_index[0], loop, pad]).reshape(NW, NCH, 128)
    dst3 = jnp.concatenate([edge_index[1], loop, pad]).reshape(NW, NCH, 128)

    att2 = (
        jnp.zeros((D, D), f32).at[:, 0].set(att_src).at[:, 1].set(att_dst)
    )
    h, ab, mx = _tc1(x, W, att2)

    a_src_p = jnp.pad(ab[:, 0], (0, NPAD - n))
    a_dst_p = jnp.pad(ab[:, 1], (0, NPAD - n))
    m = mx[0, 0] + mx[0, 1]
    m = jnp.where(m > 0.0, m, 0.2 * m)
    m16 = jnp.full((16,), m, f32)

    ex3, dens = _sc_edge_softmax(a_src_p, a_dst_p, src3, dst3, m16)

    h_pad = jnp.pad(h, ((0, NPAD - n), (0, 0)))
    zeros = jnp.zeros((NPAD, D), f32)
    outp = _sc_scatter(src3, dst3, ex3, dens, h_pad, zeros)

    return _tc2(outp[0, :n], outp[1, :n], bias.reshape(1, D))


# speedup vs baseline: 1.1699x; 1.1508x over previous
import functools

import jax
import jax.numpy as jnp
from jax import lax
from jax.experimental import pallas as pl
from jax.experimental.pallas import tpu as pltpu
from jax.experimental.pallas import tpu_sc as plsc

f32 = jnp.float32
i32 = jnp.int32

N_NODES = 10000
D = 128
NROW = 80
NPAD = NROW * 128
NC, NS = 2, 16
NW = NC * NS
NCH = 81
EPT = NCH * 128
EPAD = NW * EPT
G = 3
NGRP = NCH // G
OUT_PT = NPAD // NS
ROWB = 1000
NBLK = N_NODES // ROWB


def _tc1_body(x_ref, w_ref, att_ref, h_ref, ab_ref, mx_ref):
    i = pl.program_id(0)
    h = jnp.dot(x_ref[...], w_ref[...], preferred_element_type=f32)
    h_ref[...] = h
    ab = jnp.dot(h, att_ref[...], preferred_element_type=f32)
    ab_ref[...] = ab

    @pl.when(i == 0)
    def _():
        mx_ref[...] = jnp.full((8, 128), -jnp.inf, f32)

    bm = jnp.broadcast_to(jnp.max(ab, axis=0, keepdims=True), (8, 128))
    mx_ref[...] = jnp.maximum(mx_ref[...], bm)


_tc1 = pl.pallas_call(
    _tc1_body,
    grid=(NBLK,),
    in_specs=[
        pl.BlockSpec((ROWB, D), lambda i: (i, 0)),
        pl.BlockSpec((D, D), lambda i: (0, 0)),
        pl.BlockSpec((D, D), lambda i: (0, 0)),
    ],
    out_specs=[
        pl.BlockSpec((ROWB, D), lambda i: (i, 0)),
        pl.BlockSpec((ROWB, D), lambda i: (i, 0)),
        pl.BlockSpec((8, 128), lambda i: (0, 0)),
    ],
    out_shape=[
        jax.ShapeDtypeStruct((N_NODES, D), f32),
        jax.ShapeDtypeStruct((N_NODES, D), f32),
        jax.ShapeDtypeStruct((8, 128), f32),
    ],
)


def _sc_mesh():
    return plsc.VectorSubcoreMesh(
        core_axis_name="c", subcore_axis_name="s", num_cores=NC, num_subcores=NS
    )


@functools.partial(
    pl.kernel,
    out_type=(
        jax.ShapeDtypeStruct((NW, NCH, 128), f32),
        jax.ShapeDtypeStruct((NC, NROW, 128), f32),
    ),
    mesh=_sc_mesh(),
    compiler_params=pltpu.CompilerParams(needs_layout_passes=False, use_tc_tiling_on_sc=False),
    scratch_types=[
        pltpu.VMEM((NPAD,), f32),
        pltpu.VMEM((NPAD,), f32),
        pltpu.VMEM((NCH, 128), i32),
        pltpu.VMEM((NCH, 128), i32),
        pltpu.VMEM((NCH, 128), f32),
        pltpu.VMEM((NROW, 128), f32),
        pltpu.VMEM((16,), f32),
        pltpu.VMEM((NROW,), i32),
        pltpu.VMEM_SHARED((NROW, 128), f32),
    ],
)
def _sc_edge_softmax(asrc_h, adst_h, src_h, dst_h, m_h, ex_h, den_h,
                     asrc_v, adst_v, srcv, dstv, exv, denv, mv, idxv, dsh):
    cid = lax.axis_index("c")
    sid = lax.axis_index("s")
    wid = cid * NS + sid
    pltpu.sync_copy(asrc_h, asrc_v)
    pltpu.sync_copy(adst_h, adst_v)
    pltpu.sync_copy(m_h, mv)
    pltpu.sync_copy(src_h.at[wid], srcv)
    pltpu.sync_copy(dst_h.at[wid], dstv)

    zero16 = jnp.zeros((16,), f32)

    def zero_body(r, carry):
        for k in range(8):
            denv[r, pl.ds(k * 16, 16)] = zero16
        return carry

    lax.fori_loop(0, NROW, zero_body, 0)
    for i in range(NROW // 16):
        idxv[pl.ds(i * 16, 16)] = jnp.arange(16, dtype=i32) + (i * 16)

    @pl.when(sid == 0)
    def _():
        pltpu.sync_copy(denv, dsh)

    plsc.subcore_barrier()
    mvec = mv[...]

    def chunk(j, carry):
        for k in range(8):
            s16 = srcv[j, pl.ds(k * 16, 16)]
            d16 = dstv[j, pl.ds(k * 16, 16)]
            a = plsc.load_gather(asrc_v, [s16]) + plsc.load_gather(adst_v, [d16])
            a = jnp.where(a >= 0.0, a, 0.2 * a)
            ex = jnp.exp(a - mvec)
            exv[j, pl.ds(k * 16, 16)] = ex
            plsc.addupdate_scatter(
                denv, [lax.shift_right_logical(d16, 7), d16 & 127], ex
            )
        return carry

    lax.fori_loop(0, NCH, chunk, 0)
    pltpu.sync_copy(exv, ex_h.at[wid])
    pltpu.sync_copy(denv, dsh.at[idxv], add=True)
    plsc.subcore_barrier()

    @pl.when(sid == 0)
    def _():
        pltpu.sync_copy(dsh, den_h.at[cid])


@functools.partial(
    pl.kernel,
    out_type=jax.ShapeDtypeStruct((NC, NPAD, 128), f32),
    mesh=_sc_mesh(),
    compiler_params=pltpu.CompilerParams(needs_layout_passes=False, use_tc_tiling_on_sc=False),
    scratch_types=[
        pltpu.VMEM((NROW, 128), f32),
        pltpu.VMEM((8, 128), f32),
        pltpu.VMEM((6, 64), i32),
        pltpu.VMEM((6, 64), i32),
        pltpu.VMEM((6, 64), f32),
        pltpu.VMEM((2, 64, 128), f32),
        pltpu.VMEM_SHARED((NPAD, 128), f32),
        pltpu.SemaphoreType.DMA,
        pltpu.SemaphoreType.DMA,
    ],
)
def _sc_scatter(src_h, dst_h, ex_h, den_h, hp_h, zeros_h, outp_h,
                denv, tmpv, srcg, dstg, cfg, rows, outsh, gsem0, gsem1):
    cid = lax.axis_index("c")
    sid = lax.axis_index("s")
    wid = cid * NS + sid
    row0 = sid * OUT_PT
    pltpu.sync_copy(zeros_h.at[pl.ds(row0, OUT_PT)],
                    outsh.at[pl.ds(row0, OUT_PT)])

    pltpu.sync_copy(den_h.at[0], denv)

    def addb(s, carry):
        pltpu.sync_copy(den_h.at[1].at[pl.ds(s * 8, 8)], tmpv)
        for r in range(8):
            for k in range(8):
                denv[s * 8 + r, pl.ds(k * 16, 16)] = (
                    denv[s * 8 + r, pl.ds(k * 16, 16)]
                    + tmpv[r, pl.ds(k * 16, 16)]
                )
        return carry

    lax.fori_loop(0, NROW // 8, addb, 0)
    plsc.subcore_barrier()

    gsem = (gsem0, gsem1)

    def group(g, carry):
        pltpu.sync_copy(src_h.at[wid].at[pl.ds(g * 6, 6)], srcg)
        pltpu.sync_copy(dst_h.at[wid].at[pl.ds(g * 6, 6)], dstg)
        pltpu.sync_copy(ex_h.at[wid].at[pl.ds(g * 6, 6)], cfg)
        for hh in range(6):
            for k in range(4):
                d16 = dstg[hh, pl.ds(k * 16, 16)]
                den16 = plsc.load_gather(
                    denv, [lax.shift_right_logical(d16, 7), d16 & 127]
                )
                cfg[hh, pl.ds(k * 16, 16)] = (
                    cfg[hh, pl.ds(k * 16, 16)] / (den16 + 1e-16)
                )
        gd = [None] * 6
        for b in range(2):
            gd[b] = pltpu.async_copy(
                hp_h.at[srcg.at[b]], rows.at[b], gsem[b])
        for hh in range(6):
            b = hh & 1
            gd[hh].wait()
            rb = rows.at[b]
            cfrow = cfg.at[hh]

            def scale(e2, c2):
                for u in range(2):
                    e = e2 * 2 + u
                    cf = plsc.load_gather(cfrow, [jnp.full((16,), e, i32)])
                    for q in range(8):
                        rb[e, pl.ds(q * 16, 16)] = (
                            rb[e, pl.ds(q * 16, 16)] * cf
                        )
                return c2

            lax.fori_loop(0, 32, scale, 0)
            pltpu.sync_copy(rb, outsh.at[dstg.at[hh]], add=True)
            if hh + 2 < 6:
                gd[hh + 2] = pltpu.async_copy(
                    hp_h.at[srcg.at[hh + 2]], rows.at[b], gsem[b])
        return carry

    lax.fori_loop(0, NGRP, group, 0)
    plsc.subcore_barrier()
    pltpu.sync_copy(outsh.at[pl.ds(row0, OUT_PT)],
                    outp_h.at[cid].at[pl.ds(row0, OUT_PT)])


def _tc2_body(p0_ref, p1_ref, b_ref, o_ref):
    v = p0_ref[...] + p1_ref[...] + b_ref[...]
    o_ref[...] = jnp.where(v > 0.0, v, jnp.exp(jnp.minimum(v, 0.0)) - 1.0)


_tc2 = pl.pallas_call(
    _tc2_body,
    grid=(NBLK,),
    in_specs=[
        pl.BlockSpec((ROWB, D), lambda i: (i, 0)),
        pl.BlockSpec((ROWB, D), lambda i: (i, 0)),
        pl.BlockSpec((1, D), lambda i: (0, 0)),
    ],
    out_specs=pl.BlockSpec((ROWB, D), lambda i: (i, 0)),
    out_shape=jax.ShapeDtypeStruct((N_NODES, D), f32),
)


@jax.jit
def kernel(x, edge_index, W, att_src, att_dst, bias):
    n = x.shape[0]
    e = edge_index.shape[1]
    loop = jnp.arange(n, dtype=i32)
    pad = jnp.full((EPAD - e - n,), n, dtype=i32)
    src3 = jnp.concatenate([edge_index[0], loop, pad]).reshape(NW, NCH, 128)
    dst3 = jnp.concatenate([edge_index[1], loop, pad]).reshape(NW, NCH, 128)

    att2 = (
        jnp.zeros((D, D), f32).at[:, 0].set(att_src).at[:, 1].set(att_dst)
    )
    h, ab, mx = _tc1(x, W, att2)

    a_src_p = jnp.pad(ab[:, 0], (0, NPAD - n))
    a_dst_p = jnp.pad(ab[:, 1], (0, NPAD - n))
    m = mx[0, 0] + mx[0, 1]
    m = jnp.where(m > 0.0, m, 0.2 * m)
    m16 = jnp.full((16,), m, f32)

    ex3, dens = _sc_edge_softmax(a_src_p, a_dst_p, src3, dst3, m16)

    h_pad = jnp.pad(h, ((0, NPAD - n), (0, 0)))
    zeros = jnp.zeros((NPAD, D), f32)
    outp = _sc_scatter(
        src3.reshape(NW, NCH * 2, 64),
        dst3.reshape(NW, NCH * 2, 64),
        ex3.reshape(NW, NCH * 2, 64),
        dens, h_pad, zeros,
    )

    return _tc2(outp[0, :n], outp[1, :n], bias.reshape(1, D))


# speedup vs baseline: 1.2326x; 1.0535x over previous
import functools

import jax
import jax.numpy as jnp
from jax import lax
from jax.experimental import pallas as pl
from jax.experimental.pallas import tpu as pltpu
from jax.experimental.pallas import tpu_sc as plsc

f32 = jnp.float32
i32 = jnp.int32

N_NODES = 10000
D = 128
NROW = 80
NPAD = NROW * 128
NC, NS = 2, 16
NW = NC * NS
NCH = 81
EPT = NCH * 128
EPAD = NW * EPT
G = 3
NGRP = NCH // G
OUT_PT = NPAD // NS
ROWB = 1000
NBLK = N_NODES // ROWB


def _tc1_body(x_ref, w_ref, att_ref, h_ref, ab_ref, mx_ref):
    i = pl.program_id(0)
    h = jnp.dot(x_ref[...], w_ref[...], preferred_element_type=f32)
    h_ref[...] = h
    ab = jnp.dot(h, att_ref[...], preferred_element_type=f32)
    ab_ref[...] = ab

    @pl.when(i == 0)
    def _():
        mx_ref[...] = jnp.full((8, 128), -jnp.inf, f32)

    bm = jnp.broadcast_to(jnp.max(ab, axis=0, keepdims=True), (8, 128))
    mx_ref[...] = jnp.maximum(mx_ref[...], bm)


_tc1 = pl.pallas_call(
    _tc1_body,
    grid=(NBLK,),
    in_specs=[
        pl.BlockSpec((ROWB, D), lambda i: (i, 0)),
        pl.BlockSpec((D, D), lambda i: (0, 0)),
        pl.BlockSpec((D, D), lambda i: (0, 0)),
    ],
    out_specs=[
        pl.BlockSpec((ROWB, D), lambda i: (i, 0)),
        pl.BlockSpec((ROWB, D), lambda i: (i, 0)),
        pl.BlockSpec((8, 128), lambda i: (0, 0)),
    ],
    out_shape=[
        jax.ShapeDtypeStruct((N_NODES, D), f32),
        jax.ShapeDtypeStruct((N_NODES, D), f32),
        jax.ShapeDtypeStruct((8, 128), f32),
    ],
)


def _sc_mesh():
    return plsc.VectorSubcoreMesh(
        core_axis_name="c", subcore_axis_name="s", num_cores=NC, num_subcores=NS
    )


@functools.partial(
    pl.kernel,
    out_type=(
        jax.ShapeDtypeStruct((NW, NCH, 128), f32),
        jax.ShapeDtypeStruct((NC, NROW, 128), f32),
    ),
    mesh=_sc_mesh(),
    compiler_params=pltpu.CompilerParams(needs_layout_passes=False, use_tc_tiling_on_sc=False),
    scratch_types=[
        pltpu.VMEM((NPAD,), f32),
        pltpu.VMEM((NPAD,), f32),
        pltpu.VMEM((NCH, 128), i32),
        pltpu.VMEM((NCH, 128), i32),
        pltpu.VMEM((NCH, 128), f32),
        pltpu.VMEM((NROW, 128), f32),
        pltpu.VMEM((16,), f32),
        pltpu.VMEM((NROW,), i32),
        pltpu.VMEM_SHARED((NROW, 128), f32),
    ],
)
def _sc_edge_softmax(asrc_h, adst_h, src_h, dst_h, m_h, ex_h, den_h,
                     asrc_v, adst_v, srcv, dstv, exv, denv, mv, idxv, dsh):
    cid = lax.axis_index("c")
    sid = lax.axis_index("s")
    wid = cid * NS + sid
    pltpu.sync_copy(asrc_h, asrc_v)
    pltpu.sync_copy(adst_h, adst_v)
    pltpu.sync_copy(m_h, mv)
    pltpu.sync_copy(src_h.at[wid], srcv)
    pltpu.sync_copy(dst_h.at[wid], dstv)

    zero16 = jnp.zeros((16,), f32)

    def zero_body(r, carry):
        for k in range(8):
            denv[r, pl.ds(k * 16, 16)] = zero16
        return carry

    lax.fori_loop(0, NROW, zero_body, 0)
    for i in range(NROW // 16):
        idxv[pl.ds(i * 16, 16)] = jnp.arange(16, dtype=i32) + (i * 16)

    @pl.when(sid == 0)
    def _():
        pltpu.sync_copy(denv, dsh)

    plsc.subcore_barrier()
    mvec = mv[...]

    def chunk(j, carry):
        for k in range(8):
            s16 = srcv[j, pl.ds(k * 16, 16)]
            d16 = dstv[j, pl.ds(k * 16, 16)]
            a = plsc.load_gather(asrc_v, [s16]) + plsc.load_gather(adst_v, [d16])
            a = jnp.where(a >= 0.0, a, 0.2 * a)
            ex = jnp.exp(a - mvec)
            exv[j, pl.ds(k * 16, 16)] = ex
            plsc.addupdate_scatter(
                denv, [lax.shift_right_logical(d16, 7), d16 & 127], ex
            )
        return carry

    lax.fori_loop(0, NCH, chunk, 0)
    pltpu.sync_copy(exv, ex_h.at[wid])
    pltpu.sync_copy(denv, dsh.at[idxv], add=True)
    plsc.subcore_barrier()

    @pl.when(sid == 0)
    def _():
        pltpu.sync_copy(dsh, den_h.at[cid])


@functools.partial(
    pl.kernel,
    out_type=jax.ShapeDtypeStruct((NC, NPAD, 128), f32),
    mesh=_sc_mesh(),
    compiler_params=pltpu.CompilerParams(needs_layout_passes=False, use_tc_tiling_on_sc=False),
    scratch_types=[
        pltpu.VMEM((NROW, 128), f32),
        pltpu.VMEM((8, 128), f32),
        pltpu.VMEM((6, 64), i32),
        pltpu.VMEM((6, 64), i32),
        pltpu.VMEM((6, 64), f32),
        pltpu.VMEM((3, 64, 128), f32),
        pltpu.VMEM_SHARED((NPAD, 128), f32),
        pltpu.SemaphoreType.DMA,
        pltpu.SemaphoreType.DMA,
        pltpu.SemaphoreType.DMA,
        pltpu.SemaphoreType.DMA,
        pltpu.SemaphoreType.DMA,
        pltpu.SemaphoreType.DMA,
    ],
)
def _sc_scatter(src_h, dst_h, ex_h, den_h, hp_h, zeros_h, outp_h,
                denv, tmpv, srcg, dstg, cfg, rows, outsh,
                gsem0, gsem1, gsem2, ssem0, ssem1, ssem2):
    cid = lax.axis_index("c")
    sid = lax.axis_index("s")
    wid = cid * NS + sid
    row0 = sid * OUT_PT
    pltpu.sync_copy(zeros_h.at[pl.ds(row0, OUT_PT)],
                    outsh.at[pl.ds(row0, OUT_PT)])

    pltpu.sync_copy(den_h.at[0], denv)

    def addb(s, carry):
        pltpu.sync_copy(den_h.at[1].at[pl.ds(s * 8, 8)], tmpv)
        for r in range(8):
            for k in range(8):
                denv[s * 8 + r, pl.ds(k * 16, 16)] = (
                    denv[s * 8 + r, pl.ds(k * 16, 16)]
                    + tmpv[r, pl.ds(k * 16, 16)]
                )
        return carry

    lax.fori_loop(0, NROW // 8, addb, 0)
    plsc.subcore_barrier()

    gsem = (gsem0, gsem1, gsem2)
    ssem = (ssem0, ssem1, ssem2)

    def group(g, carry):
        pltpu.sync_copy(src_h.at[wid].at[pl.ds(g * 6, 6)], srcg)
        pltpu.sync_copy(dst_h.at[wid].at[pl.ds(g * 6, 6)], dstg)
        pltpu.sync_copy(ex_h.at[wid].at[pl.ds(g * 6, 6)], cfg)
        for hh in range(6):
            for k in range(4):
                d16 = dstg[hh, pl.ds(k * 16, 16)]
                den16 = plsc.load_gather(
                    denv, [lax.shift_right_logical(d16, 7), d16 & 127]
                )
                cfg[hh, pl.ds(k * 16, 16)] = (
                    cfg[hh, pl.ds(k * 16, 16)] / (den16 + 1e-16)
                )
        gd = [None] * 6
        sd = [None] * 6
        for b in range(2):
            gd[b] = pltpu.async_copy(
                hp_h.at[srcg.at[b]], rows.at[b], gsem[b])
        for hh in range(6):
            s = hh % 3
            if hh >= 2:
                sd[hh - 2].wait()
            if hh + 1 < 6 and hh >= 1:
                s1 = (hh + 1) % 3
                gd[hh + 1] = pltpu.async_copy(
                    hp_h.at[srcg.at[hh + 1]], rows.at[s1], gsem[s1])
            gd[hh].wait()
            rb = rows.at[s]
            cfrow = cfg.at[hh]

            def scale(e2, c2):
                for u in range(2):
                    e = e2 * 2 + u
                    cf = plsc.load_gather(cfrow, [jnp.full((16,), e, i32)])
                    for q in range(8):
                        rb[e, pl.ds(q * 16, 16)] = (
                            rb[e, pl.ds(q * 16, 16)] * cf
                        )
                return c2

            lax.fori_loop(0, 32, scale, 0)
            sd[hh] = pltpu.async_copy(
                rb, outsh.at[dstg.at[hh]], ssem[s], add=True)
        sd[4].wait()
        sd[5].wait()
        return carry

    lax.fori_loop(0, NGRP, group, 0)
    plsc.subcore_barrier()
    pltpu.sync_copy(outsh.at[pl.ds(row0, OUT_PT)],
                    outp_h.at[cid].at[pl.ds(row0, OUT_PT)])


def _tc2_body(p0_ref, p1_ref, b_ref, o_ref):
    v = p0_ref[...] + p1_ref[...] + b_ref[...]
    o_ref[...] = jnp.where(v > 0.0, v, jnp.exp(jnp.minimum(v, 0.0)) - 1.0)


_tc2 = pl.pallas_call(
    _tc2_body,
    grid=(NBLK,),
    in_specs=[
        pl.BlockSpec((ROWB, D), lambda i: (i, 0)),
        pl.BlockSpec((ROWB, D), lambda i: (i, 0)),
        pl.BlockSpec((1, D), lambda i: (0, 0)),
    ],
    out_specs=pl.BlockSpec((ROWB, D), lambda i: (i, 0)),
    out_shape=jax.ShapeDtypeStruct((N_NODES, D), f32),
)


@jax.jit
def kernel(x, edge_index, W, att_src, att_dst, bias):
    n = x.shape[0]
    e = edge_index.shape[1]
    loop = jnp.arange(n, dtype=i32)
    pad = jnp.full((EPAD - e - n,), n, dtype=i32)
    src3 = jnp.concatenate([edge_index[0], loop, pad]).reshape(NW, NCH, 128)
    dst3 = jnp.concatenate([edge_index[1], loop, pad]).reshape(NW, NCH, 128)

    att2 = (
        jnp.zeros((D, D), f32).at[:, 0].set(att_src).at[:, 1].set(att_dst)
    )
    h, ab, mx = _tc1(x, W, att2)

    a_src_p = jnp.pad(ab[:, 0], (0, NPAD - n))
    a_dst_p = jnp.pad(ab[:, 1], (0, NPAD - n))
    m = mx[0, 0] + mx[0, 1]
    m = jnp.where(m > 0.0, m, 0.2 * m)
    m16 = jnp.full((16,), m, f32)

    ex3, dens = _sc_edge_softmax(a_src_p, a_dst_p, src3, dst3, m16)

    h_pad = jnp.pad(h, ((0, NPAD - n), (0, 0)))
    zeros = jnp.zeros((NPAD, D), f32)
    outp = _sc_scatter(
        src3.reshape(NW, NCH * 2, 64),
        dst3.reshape(NW, NCH * 2, 64),
        ex3.reshape(NW, NCH * 2, 64),
        dens, h_pad, zeros,
    )

    return _tc2(outp[0, :n], outp[1, :n], bias.reshape(1, D))
